# Initial kernel scaffold; baseline (speedup 1.0000x reference)
import jax
import jax.numpy as jnp
from jax.experimental import pallas as pl


def kernel(query, key, value, combined_shifts, alpha):
    raise NotImplementedError("write your pallas kernel here")


# speedup vs baseline: 3.7420x; 3.7420x over previous
import functools

import jax
import jax.numpy as jnp
from jax import lax
from jax.experimental import pallas as pl
from jax.experimental.pallas import tpu as pltpu

N_HASHES = 2
N_HEADS = 8
PADDED = 4096
BLOCK = 64
DPH = 32
CDIM = 3
D = DPH + CDIM


def _hash_body(q_ref, k_ref, al_ref, cs_ref, qh_ref, kh_ref):
    q = q_ref[0]
    k = k_ref[0]
    al = al_ref[0]
    cs = cs_ref[0].astype(jnp.float32)
    qh = jax.lax.dot_general(q, al, (((1,), (0,)), ((), ())),
                             preferred_element_type=jnp.float32, precision=jax.lax.Precision.HIGHEST)
    kh = jax.lax.dot_general(k, al, (((1,), (0,)), ((), ())),
                             preferred_element_type=jnp.float32, precision=jax.lax.Precision.HIGHEST)
    mx = jnp.maximum(jnp.max(qh, axis=0), jnp.max(kh, axis=0))
    mn = jnp.minimum(jnp.min(qh, axis=0), jnp.min(kh, axis=0))
    shift = cs * (mx - mn)[None, :]
    qh_ref[0] = qh + shift
    kh_ref[0] = kh + shift


def _hash_stage(query, key, alpha, cs_t):
    out = pl.pallas_call(
        _hash_body,
        grid=(N_HEADS,),
        in_specs=[
            pl.BlockSpec((1, PADDED, D), lambda h: (h, 0, 0)),
            pl.BlockSpec((1, PADDED, D), lambda h: (h, 0, 0)),
            pl.BlockSpec((1, D, N_HASHES), lambda h: (h, 0, 0)),
            pl.BlockSpec((1, PADDED, N_HASHES), lambda h: (h, 0, 0)),
        ],
        out_specs=[
            pl.BlockSpec((1, PADDED, N_HASHES), lambda h: (h, 0, 0)),
            pl.BlockSpec((1, PADDED, N_HASHES), lambda h: (h, 0, 0)),
        ],
        out_shape=[
            jax.ShapeDtypeStruct((N_HEADS, PADDED, N_HASHES), jnp.float32),
            jax.ShapeDtypeStruct((N_HEADS, PADDED, N_HASHES), jnp.float32),
        ],
    )(query, key, alpha, cs_t)
    return out


BLOCKS_PER_STEP = 8
ROWS_PER_STEP = BLOCKS_PER_STEP * BLOCK


def _attn_body(q_ref, k_ref, v_ref, o_ref):
    q = q_ref[0, 0].reshape(BLOCKS_PER_STEP, BLOCK, D)
    k = k_ref[0, 0].reshape(BLOCKS_PER_STEP, BLOCK, D)
    v = v_ref[0, 0].reshape(BLOCKS_PER_STEP, BLOCK, DPH)
    qn = jnp.sum(q * q, axis=-1)
    kn = jnp.sum(k * k, axis=-1)
    qk = jax.lax.dot_general(q, k, (((2,), (2,)), ((0,), (0,))),
                             preferred_element_type=jnp.float32, precision=jax.lax.Precision.HIGHEST)
    d2 = qn[:, :, None] + kn[:, None, :] - 2.0 * qk
    w = jnp.exp(-0.5 * d2)
    o = jax.lax.dot_general(w, v, (((2,), (1,)), ((0,), (0,))),
                            preferred_element_type=jnp.float32, precision=jax.lax.Precision.HIGHEST)
    o_ref[0, 0] = o


def _attn_stage(sq, sk, sv):
    n_steps = PADDED // ROWS_PER_STEP
    out = pl.pallas_call(
        _attn_body,
        grid=(N_HASHES, N_HEADS, n_steps),
        in_specs=[
            pl.BlockSpec((1, 1, ROWS_PER_STEP, D), lambda a, h, b: (a, h, b, 0)),
            pl.BlockSpec((1, 1, ROWS_PER_STEP, D), lambda a, h, b: (a, h, b, 0)),
            pl.BlockSpec((1, 1, ROWS_PER_STEP, DPH), lambda a, h, b: (a, h, b, 0)),
        ],
        out_specs=pl.BlockSpec((1, 1, BLOCKS_PER_STEP, BLOCK, DPH),
                               lambda a, h, b: (a, h, b, 0, 0)),
        out_shape=jax.ShapeDtypeStruct(
            (N_HASHES, N_HEADS, PADDED // BLOCK, BLOCK, DPH), jnp.float32),
    )(sq, sk, sv)
    return out


def kernel(query, key, value, combined_shifts, alpha):
    q_hashed = jnp.einsum('hpd,hdn->hpn', query, alpha).transpose(2, 0, 1)
    k_hashed = jnp.einsum('hpd,hdn->hpn', key, alpha).transpose(2, 0, 1)
    max_hash_shift = jnp.maximum(q_hashed.max(-1, keepdims=True),
                                 k_hashed.max(-1, keepdims=True))
    min_hash_shift = jnp.minimum(q_hashed.min(-1, keepdims=True),
                                 k_hashed.min(-1, keepdims=True))
    hash_shift = max_hash_shift - min_hash_shift
    cs = combined_shifts.astype(jnp.float32) * hash_shift
    q_keys = q_hashed + cs
    k_keys = k_hashed + cs
    q_pos = jnp.argsort(q_keys, axis=-1)
    k_pos = jnp.argsort(k_keys, axis=-1)
    q_exp = jnp.broadcast_to(query[None], (N_HASHES,) + query.shape)
    sq = jnp.take_along_axis(q_exp, q_pos[..., None], axis=2)
    k_exp = jnp.broadcast_to(key[None], (N_HASHES,) + key.shape)
    sk = jnp.take_along_axis(k_exp, k_pos[..., None], axis=2)
    v_exp = jnp.broadcast_to(value[None], (N_HASHES,) + value.shape)
    sv = jnp.take_along_axis(v_exp, k_pos[..., None], axis=2)
    return _attn_stage(sq, sk, sv)
